# Initial kernel scaffold; baseline (speedup 1.0000x reference)
import jax
import jax.numpy as jnp
from jax.experimental import pallas as pl


def kernel(x, edge_index, batch, emb, W_l, b_l, W_r, w1, w2, lin1_W, lin1_b, lin3_W, lin3_b):
    raise NotImplementedError("write your pallas kernel here")


# speedup vs baseline: 3.8788x; 3.8788x over previous
import functools

import jax
import jax.numpy as jnp
from jax import lax
from jax.experimental import pallas as pl
from jax.experimental.pallas import tpu as pltpu
from jax.experimental.pallas import tpu_sc as plsc

N = 10000
NPAD = 10240
E = 320000
G = 64
D = 128
VOCAB = 20215
NW = 32
RPW = NPAD // NW
EPW = E // NW
ECH = 128
NFULL = EPW // ECH
ETAIL = EPW - NFULL * ECH
ROWB = 256
PADC = 768
NCOL = NPAD + 2 * PADC
BAND = 1792
RWIN = 1280
SENT_ROW = 1000000
SENT_COL = 2000000
NEG = float("-inf")


def _sc_gather(emb, xpad):
    mesh = plsc.VectorSubcoreMesh(core_axis_name="c", subcore_axis_name="s")
    CH = 80

    @functools.partial(
        pl.kernel, mesh=mesh,
        out_type=jax.ShapeDtypeStruct((NPAD, D), jnp.float32),
        scratch_types=[
            pltpu.VMEM((CH,), jnp.int32),
            pltpu.VMEM((CH, D), jnp.float32),
            pltpu.SemaphoreType.DMA,
        ],
    )
    def k(emb_hbm, idx_hbm, out_hbm, idx_v, rows_v, sem):
        wid = lax.axis_index("s") * 2 + lax.axis_index("c")
        base = wid * RPW
        for j in range(RPW // CH):
            off = base + j * CH
            pltpu.sync_copy(idx_hbm.at[pl.ds(off, CH)], idx_v)
            pltpu.async_copy(emb_hbm.at[idx_v], rows_v, sem).wait()
            pltpu.sync_copy(rows_v, out_hbm.at[pl.ds(off, CH)])

    return k(emb, xpad)


def _sc_edges(h, src, dst, zrows, ones_f):
    mesh = plsc.VectorSubcoreMesh(core_axis_name="c", subcore_axis_name="s")
    ZR = NPAD // 16

    @functools.partial(
        pl.kernel, mesh=mesh,
        out_type=(jax.ShapeDtypeStruct((2, NPAD, D), jnp.float32),
                  jax.ShapeDtypeStruct((2, NPAD, D), jnp.float32)),
        scratch_types=[
            pltpu.VMEM((ECH,), jnp.int32),
            pltpu.VMEM((ECH,), jnp.int32),
            pltpu.VMEM((ECH, D), jnp.float32),
            pltpu.VMEM((ETAIL,), jnp.int32),
            pltpu.VMEM((ETAIL,), jnp.int32),
            pltpu.VMEM((ETAIL, D), jnp.float32),
            pltpu.VMEM_SHARED((NPAD, D), jnp.float32),
            pltpu.SemaphoreType.DMA,
        ],
    )
    def k(h_hbm, src_hbm, dst_hbm, zrows_hbm, ones_hbm, agg_hbm, cnt_hbm,
          src_v, dst_v, rows_v, srct_v, dstt_v, rowst_v, agg_sh, sem):
        cid = lax.axis_index("c")
        sid = lax.axis_index("s")
        wid = sid * 2 + cid
        ebase = wid * EPW
        toff = ebase + NFULL * ECH

        pltpu.sync_copy(zrows_hbm, agg_sh.at[pl.ds(sid * ZR, ZR)])
        plsc.subcore_barrier()

        def body(i, carry):
            off = ebase + i * ECH
            pltpu.sync_copy(src_hbm.at[pl.ds(off, ECH)], src_v)
            pltpu.sync_copy(dst_hbm.at[pl.ds(off, ECH)], dst_v)
            pltpu.async_copy(h_hbm.at[src_v], rows_v, sem).wait()
            pltpu.sync_copy(rows_v, agg_sh.at[dst_v], add=True)
            return carry

        lax.fori_loop(0, NFULL, body, 0)
        pltpu.sync_copy(src_hbm.at[pl.ds(toff, ETAIL)], srct_v)
        pltpu.sync_copy(dst_hbm.at[pl.ds(toff, ETAIL)], dstt_v)
        pltpu.async_copy(h_hbm.at[srct_v], rowst_v, sem).wait()
        pltpu.sync_copy(rowst_v, agg_sh.at[dstt_v], add=True)
        plsc.subcore_barrier()
        pltpu.sync_copy(agg_sh.at[pl.ds(sid * ZR, ZR)],
                        agg_hbm.at[cid, pl.ds(sid * ZR, ZR)])
        plsc.subcore_barrier()

        pltpu.sync_copy(zrows_hbm, agg_sh.at[pl.ds(sid * ZR, ZR)])
        pltpu.sync_copy(ones_hbm, rows_v)
        plsc.subcore_barrier()

        def body2(i, carry):
            off = ebase + i * ECH
            pltpu.sync_copy(dst_hbm.at[pl.ds(off, ECH)], dst_v)
            pltpu.sync_copy(rows_v, agg_sh.at[dst_v], add=True)
            return carry

        lax.fori_loop(0, NFULL, body2, 0)
        pltpu.sync_copy(dst_hbm.at[pl.ds(toff, ETAIL)], dstt_v)
        pltpu.sync_copy(rows_v.at[pl.ds(0, ETAIL)], agg_sh.at[dstt_v],
                        add=True)
        plsc.subcore_barrier()
        pltpu.sync_copy(agg_sh.at[pl.ds(sid * ZR, ZR)],
                        cnt_hbm.at[cid, pl.ds(sid * ZR, ZR)])

    return k(h, src, dst, zrows, ones_f)


def _dense_body(h_ref, a0_ref, a1_ref, c0_ref, c1_ref, seg_ref,
                wl_ref, bl_ref, wr_ref, w1_ref, w2_ref,
                hs_ref, s1_ref, p2_ref, cnts_ref):
    i = pl.program_id(0)
    cnt = c0_ref[:, :1] + c1_ref[:, :1]
    agg = (a0_ref[...] + a1_ref[...]) / jnp.maximum(cnt, 1.0)
    h = h_ref[...]
    hs = jax.lax.dot_general(agg, wl_ref[...], (((1,), (1,)), ((), ())),
                             preferred_element_type=jnp.float32)
    hs = hs + bl_ref[...] + jax.lax.dot_general(
        h, wr_ref[...], (((1,), (1,)), ((), ())),
        preferred_element_type=jnp.float32)
    hs = jnp.maximum(hs, 0.0)
    hs_ref[...] = hs
    w1 = w1_ref[...]
    w2 = w2_ref[...]
    n1 = jax.lax.rsqrt(jnp.sum(w1 * w1))
    n2 = jax.lax.rsqrt(jnp.sum(w2 * w2))
    s1_ref[...] = jnp.tanh(jax.lax.dot_general(
        hs, w1, (((1,), (1,)), ((), ())), preferred_element_type=jnp.float32) * n1)
    p2_ref[...] = jax.lax.dot_general(
        hs, w2, (((1,), (1,)), ((), ())), preferred_element_type=jnp.float32) * n2
    gid = jax.lax.broadcasted_iota(jnp.int32, (1, G), 1)
    onehot = (seg_ref[...] == gid).astype(jnp.int32)

    @pl.when(i == 0)
    def _():
        cnts_ref[...] = jnp.zeros_like(cnts_ref)

    cnts_ref[...] += jnp.sum(onehot, axis=0, keepdims=True)


def _tc_dense(h, a0, a1, c0, c1, segrow, W_l, b_l, W_r, w1, w2):
    nb = NPAD // ROWB
    return pl.pallas_call(
        _dense_body,
        grid=(nb,),
        in_specs=[
            pl.BlockSpec((ROWB, D), lambda i: (i, 0)),
            pl.BlockSpec((ROWB, D), lambda i: (i, 0)),
            pl.BlockSpec((ROWB, D), lambda i: (i, 0)),
            pl.BlockSpec((ROWB, D), lambda i: (i, 0)),
            pl.BlockSpec((ROWB, D), lambda i: (i, 0)),
            pl.BlockSpec((ROWB, 1), lambda i: (i, 0)),
            pl.BlockSpec((D, D), lambda i: (0, 0)),
            pl.BlockSpec((1, D), lambda i: (0, 0)),
            pl.BlockSpec((D, D), lambda i: (0, 0)),
            pl.BlockSpec((1, D), lambda i: (0, 0)),
            pl.BlockSpec((1, D), lambda i: (0, 0)),
        ],
        out_specs=[
            pl.BlockSpec((ROWB, D), lambda i: (i, 0)),
            pl.BlockSpec((ROWB, 1), lambda i: (i, 0)),
            pl.BlockSpec((ROWB, 1), lambda i: (i, 0)),
            pl.BlockSpec((1, G), lambda i: (0, 0)),
        ],
        out_shape=[
            jax.ShapeDtypeStruct((NPAD, D), jnp.float32),
            jax.ShapeDtypeStruct((NPAD, 1), jnp.float32),
            jax.ShapeDtypeStruct((NPAD, 1), jnp.float32),
            jax.ShapeDtypeStruct((1, G), jnp.int32),
        ],
    )(h, a0, a1, c0, c1, segrow, W_l, b_l, W_r, w1, w2)


def _k_from_counts(c):
    k1 = jnp.maximum((4 * c + 4) // 5, 1)
    k2 = jnp.maximum((4 * k1 + 4) // 5, 1)
    return k1, k2


def _rank_block(i, srow, seg_row, scol, seg_col):
    posr = i * ROWB + jax.lax.broadcasted_iota(jnp.int32, (ROWB, 1), 0)
    posc = i * ROWB - PADC + jax.lax.broadcasted_iota(jnp.int32, (1, BAND), 1)
    seg_eq = seg_row == seg_col
    gt = scol > srow
    tie = (scol == srow) & (posc < posr)
    return jnp.sum((seg_eq & (gt | tie)).astype(jnp.int32), axis=1,
                   keepdims=True)


def _krow(seg_row, kvec):
    gid = jax.lax.broadcasted_iota(jnp.int32, (1, G), 1)
    return jnp.sum(jnp.where(seg_row == gid, kvec, 0), axis=1, keepdims=True)


def _rank1_body(s1_ref, seg_ref, s1c_ref, segc_ref, cnts_ref, m1_ref, v1_ref):
    i = pl.program_id(0)
    srow = s1_ref[...]
    scol = s1c_ref[:, pl.ds(i * ROWB, BAND)]
    segc = segc_ref[:, pl.ds(i * ROWB, BAND)]
    rank = _rank_block(i, srow, seg_ref[...], scol, segc)
    k1, _ = _k_from_counts(cnts_ref[...])
    mask = rank < _krow(seg_ref[...], k1)
    m1_ref[...] = mask.astype(jnp.float32)
    v1_ref[...] = jnp.where(mask, srow, 0.0)


def _tc_rank1(s1, segrow, s1col, segcol, counts):
    nb = NPAD // ROWB
    return pl.pallas_call(
        _rank1_body,
        grid=(nb,),
        in_specs=[
            pl.BlockSpec((ROWB, 1), lambda i: (i, 0)),
            pl.BlockSpec((ROWB, 1), lambda i: (i, 0)),
            pl.BlockSpec((1, NCOL), lambda i: (0, 0)),
            pl.BlockSpec((1, NCOL), lambda i: (0, 0)),
            pl.BlockSpec((1, G), lambda i: (0, 0)),
        ],
        out_specs=[
            pl.BlockSpec((ROWB, 1), lambda i: (i, 0)),
            pl.BlockSpec((ROWB, 1), lambda i: (i, 0)),
        ],
        out_shape=[
            jax.ShapeDtypeStruct((NPAD, 1), jnp.float32),
            jax.ShapeDtypeStruct((NPAD, 1), jnp.float32),
        ],
    )(s1, segrow, s1col, segcol, counts)


def _rank2_body(v1_ref, m1_ref, p2_ref, seg_ref, v1c_ref, m1c_ref, p2c_ref,
                segc_ref, cnts_ref, m2_ref, v2_ref):
    i = pl.program_id(0)
    s2row = jnp.where(m1_ref[...] > 0.0, jnp.tanh(v1_ref[...] * p2_ref[...]),
                      NEG)
    v1c = v1c_ref[:, pl.ds(i * ROWB, BAND)]
    m1c = m1c_ref[:, pl.ds(i * ROWB, BAND)]
    p2c = p2c_ref[:, pl.ds(i * ROWB, BAND)]
    s2col = jnp.where(m1c > 0.0, jnp.tanh(v1c * p2c), NEG)
    segc = segc_ref[:, pl.ds(i * ROWB, BAND)]
    rank = _rank_block(i, s2row, seg_ref[...], s2col, segc)
    _, k2 = _k_from_counts(cnts_ref[...])
    mask = rank < _krow(seg_ref[...], k2)
    m2_ref[...] = mask.astype(jnp.float32)
    v2_ref[...] = jnp.where(mask, s2row, 0.0)


def _tc_rank2(v1, m1, p2, segrow, v1col, m1col, p2col, segcol, counts):
    nb = NPAD // ROWB
    full = pl.BlockSpec((1, NCOL), lambda i: (0, 0))
    blk = pl.BlockSpec((ROWB, 1), lambda i: (i, 0))
    return pl.pallas_call(
        _rank2_body,
        grid=(nb,),
        in_specs=[blk, blk, blk, blk, full, full, full, full,
                  pl.BlockSpec((1, G), lambda i: (0, 0))],
        out_specs=[blk, blk],
        out_shape=[
            jax.ShapeDtypeStruct((NPAD, 1), jnp.float32),
            jax.ShapeDtypeStruct((NPAD, 1), jnp.float32),
        ],
    )(v1, m1, p2, segrow, v1col, m1col, p2col, segcol, counts)


def _readout_body(cnts_ref, hs_ref, v1_ref, m1_ref, v2_ref, m2_ref, out_ref):
    g = pl.program_id(0)

    def body(gg, acc):
        return acc + jnp.where(gg < g, cnts_ref[0, gg], 0)

    start = lax.fori_loop(0, G, body, jnp.int32(0))
    cnt = cnts_ref[0, g]
    k1 = jnp.maximum((4 * cnt + 4) // 5, 1)
    k2 = jnp.maximum((4 * k1 + 4) // 5, 1)
    aligned = jnp.minimum((start // 8) * 8, NPAD - RWIN)
    rows = hs_ref[pl.ds(aligned, RWIN), :]
    v1 = v1_ref[pl.ds(aligned, RWIN), :]
    m1 = m1_ref[pl.ds(aligned, RWIN), :]
    v2 = v2_ref[pl.ds(aligned, RWIN), :]
    m2 = m2_ref[pl.ds(aligned, RWIN), :]
    pos = aligned + jax.lax.broadcasted_iota(jnp.int32, (RWIN, 1), 0)
    inseg = (pos >= start) & (pos < start + cnt)
    g1 = rows * v1
    sel1 = inseg & (m1 > 0.0)
    x1max = jnp.max(jnp.where(sel1, g1, NEG), axis=0, keepdims=True)
    x1mean = jnp.sum(jnp.where(sel1, g1, 0.0), axis=0,
                     keepdims=True) / k1.astype(jnp.float32)
    g2 = g1 * v2
    sel2 = inseg & (m2 > 0.0)
    x2max = jnp.max(jnp.where(sel2, g2, NEG), axis=0, keepdims=True)
    x2mean = jnp.sum(jnp.where(sel2, g2, 0.0), axis=0,
                     keepdims=True) / k2.astype(jnp.float32)
    out_ref[...] = jnp.concatenate([x1max + x2max, x1mean + x2mean],
                                   axis=1).reshape(1, 1, 2 * D)


def _tc_readout(counts, hs, v1, m1, v2, m2):
    fullcol = pl.BlockSpec((NPAD, 1), lambda g: (0, 0))
    return pl.pallas_call(
        _readout_body,
        grid=(G,),
        in_specs=[
            pl.BlockSpec(memory_space=pltpu.SMEM),
            pl.BlockSpec((NPAD, D), lambda g: (0, 0)),
            fullcol, fullcol, fullcol, fullcol,
        ],
        out_specs=pl.BlockSpec((1, 1, 2 * D), lambda g: (g, 0, 0)),
        out_shape=jax.ShapeDtypeStruct((G, 1, 2 * D), jnp.float32),
    )(counts, hs, v1, m1, v2, m2)


def _mlp_body(x_ref, w1_ref, b1_ref, w3_ref, b3_ref, out_ref):
    y = jax.lax.dot_general(x_ref[...], w1_ref[...], (((1,), (1,)), ((), ())),
                            preferred_element_type=jnp.float32)
    y = jnp.maximum(y + b1_ref[...], 0.0)
    z = jnp.sum(y * w3_ref[...], axis=1, keepdims=True) + b3_ref[0, 0]
    out_ref[...] = jax.nn.sigmoid(z)


def _tc_mlp(x12, lin1_W, lin1_b, lin3_W, lin3_b):
    return pl.pallas_call(
        _mlp_body,
        in_specs=[
            pl.BlockSpec((G, 2 * D), lambda: (0, 0)),
            pl.BlockSpec((G, 2 * D), lambda: (0, 0)),
            pl.BlockSpec((1, G), lambda: (0, 0)),
            pl.BlockSpec((1, G), lambda: (0, 0)),
            pl.BlockSpec(memory_space=pltpu.SMEM),
        ],
        out_specs=pl.BlockSpec((G, 1), lambda: (0, 0)),
        out_shape=jax.ShapeDtypeStruct((G, 1), jnp.float32),
    )(x12, lin1_W, lin1_b, lin3_W, lin3_b)


def kernel(x, edge_index, batch, emb, W_l, b_l, W_r, w1, w2, lin1_W, lin1_b,
           lin3_W, lin3_b):
    xpad = jnp.pad(x[:, 0], (0, NPAD - N))
    h = _sc_gather(emb, xpad)
    src = edge_index[0]
    dst = edge_index[1]
    zrows = jnp.zeros((NPAD // 16, D), jnp.float32)
    ones_f = jnp.ones((ECH, D), jnp.float32)
    aggp, cntp = _sc_edges(h, src, dst, zrows, ones_f)

    segrow = jnp.pad(batch, (0, NPAD - N),
                     constant_values=SENT_ROW).reshape(NPAD, 1)
    hs, s1, p2, counts = _tc_dense(h, aggp[0], aggp[1], cntp[0], cntp[1],
                                   segrow, W_l, b_l.reshape(1, D), W_r,
                                   w1.reshape(1, D), w2.reshape(1, D))
    segcol = jnp.pad(segrow.reshape(1, NPAD), ((0, 0), (PADC, PADC)),
                     constant_values=SENT_COL)
    s1col = jnp.pad(s1.reshape(1, NPAD), ((0, 0), (PADC, PADC)))
    m1, v1 = _tc_rank1(s1, segrow, s1col, segcol, counts)
    v1col = jnp.pad(v1.reshape(1, NPAD), ((0, 0), (PADC, PADC)))
    m1col = jnp.pad(m1.reshape(1, NPAD), ((0, 0), (PADC, PADC)))
    p2col = jnp.pad(p2.reshape(1, NPAD), ((0, 0), (PADC, PADC)))
    m2, v2 = _tc_rank2(v1, m1, p2, segrow, v1col, m1col, p2col, segcol,
                       counts)
    x12 = _tc_readout(counts, hs, v1, m1, v2, m2).reshape(G, 2 * D)
    out = _tc_mlp(x12, lin1_W, lin1_b.reshape(1, G), lin3_W,
                  lin3_b.reshape(1, 1))
    return out.reshape(G)
